# baseline (speedup 1.0000x reference)
import functools

import jax
import jax.numpy as jnp
import numpy as np
from jax import lax
from jax.experimental import pallas as pl
from jax.experimental.pallas import tpu as pltpu
from jax.experimental.pallas import tpu_sc as plsc

N = 10000
E = 320000
D = 128
H = 8
DH = D // H

NC = 2
NS = 16
NW = NC * NS
EPW = E // NW
EB = 32
NBLK = E // EB
NBPW = NBLK // NW
NP = 10240
NZ = NP // 8
RPT = NP // NS
ZPT = NZ // NS

_PREC = lax.Precision.HIGHEST


def _dot(a, b):
    return jnp.dot(a, b, preferred_element_type=jnp.float32, precision=_PREC)


def _ln_rows(x, g, b):
    mu = jnp.mean(x, axis=-1, keepdims=True)
    xc = x - mu
    var = jnp.mean(xc * xc, axis=-1, keepdims=True)
    return xc * lax.rsqrt(var + 1e-5) * g + b


def _proj_body(h_ref, wq_ref, wk_ref, wv_ref, q_ref, k_ref, v_ref):
    hb = h_ref[...]
    q_ref[...] = _dot(hb, wq_ref[...])
    k_ref[...] = _dot(hb, wk_ref[...]) * (1.0 / np.sqrt(DH))
    v_ref[...] = _dot(hb, wv_ref[...])


def _proj(h, wq, wk, wv):
    blk = 2000
    grid = N // blk
    wspec = pl.BlockSpec((D, D), lambda i: (0, 0))
    rspec = pl.BlockSpec((blk, D), lambda i: (i, 0))
    return pl.pallas_call(
        _proj_body,
        grid=(grid,),
        in_specs=[rspec, wspec, wspec, wspec],
        out_specs=[rspec, rspec, rspec],
        out_shape=[jax.ShapeDtypeStruct((N, D), jnp.float32)] * 3,
    )(h, wq, wk, wv)


def _pe_body(e_ref, we_ref, pe_ref):
    pe_ref[...] = _dot(e_ref[...], we_ref[...])


def _pe(e, we):
    blk = 2560
    grid = E // blk
    return pl.pallas_call(
        _pe_body,
        grid=(grid,),
        in_specs=[pl.BlockSpec((blk, D), lambda i: (i, 0)),
                  pl.BlockSpec((D, D), lambda i: (0, 0))],
        out_specs=pl.BlockSpec((blk, D), lambda i: (i, 0)),
        out_shape=jax.ShapeDtypeStruct((E, D), jnp.float32),
    )(e, we)


def _sc_edge_body(kh, qh, vh, pe, sdfr, eout, accout, zaccout,
                  idx_sd, idx_z, krows, qrows, vrows, perows, erows,
                  contrib, contribz, zbuf, acc, zacc, sem):
    cid = lax.axis_index("c")
    sid = lax.axis_index("s")
    wid = cid * NS + sid

    zv = jnp.zeros((16,), jnp.float32)
    lanes = lax.iota(jnp.int32, 16)

    def zbrow(i, carry):
        for j in range(D // 16):
            zbuf[i, pl.ds(j * 16, 16)] = zv
        return carry

    lax.fori_loop(0, EB, zbrow, 0)

    def zaccrow(b, carry):
        pltpu.sync_copy(zbuf, acc.at[pl.ds(sid * RPT + b * EB, EB)])
        return carry

    lax.fori_loop(0, RPT // EB, zaccrow, 0)
    for zoff in (0, 32, 48):
        pltpu.sync_copy(zbuf, zacc.at[pl.ds(sid * ZPT + zoff, EB)])
    plsc.subcore_barrier()

    gdn = lax.GatherDimensionNumbers(offset_dims=(), collapsed_slice_dims=(0,),
                                     start_index_map=(0,))
    bfly_idx = [(lanes ^ k)[:, None] for k in (8, 4, 2, 1)]

    def _lane_allsum(v):
        for idx in bfly_idx:
            v = v + lax.gather(v, idx, gdn, slice_sizes=(1,),
                               mode=lax.GatherScatterMode.PROMISE_IN_BOUNDS)
        return v

    def _bcast_lane(v, l):
        return lax.gather(v, jnp.full((16, 1), l, jnp.int32), gdn,
                          slice_sizes=(1,),
                          mode=lax.GatherScatterMode.PROMISE_IN_BOUNDS)

    def do_block(blk, carry):
        base = blk * EB
        pltpu.sync_copy(sdfr.at[blk], idx_sd)

        for off in (0, 16):
            c1 = pltpu.async_copy(kh.at[idx_sd.at[0, pl.ds(off, 16)]],
                                  krows, sem)
            c2 = pltpu.async_copy(qh.at[idx_sd.at[1, pl.ds(off, 16)]],
                                  qrows, sem)
            c3 = pltpu.async_copy(vh.at[idx_sd.at[0, pl.ds(off, 16)]],
                                  vrows, sem)
            if off == 0:
                pltpu.sync_copy(pe.at[pl.ds(base, EB)], perows)
            dv = idx_sd[1, pl.ds(off, 16)]
            idx_z[pl.ds(off, 16)] = lax.shift_right_logical(dv, 3)
            slotf = lax.bitwise_and(dv, 7).astype(jnp.float32)
            c1.wait()
            c2.wait()
            c3.wait()
            for l in range(16):
                row = off + l
                svec = zv
                for hh in range(H):
                    sl = pl.ds(hh * DH, DH)
                    sc = krows[l, sl] * qrows[l, sl] * perows[row, sl]
                    erows[row, sl] = sc
                    tot = _lane_allsum(sc)
                    sv = jnp.exp(jnp.minimum(jnp.maximum(tot, -5.0), 5.0))
                    contrib[row, sl] = vrows[l, sl] * sv
                    svec = jnp.where(lanes == hh, sv, svec)
                sb = _bcast_lane(slotf, l)
                for k in range(8):
                    ind = jnp.maximum(1.0 - jnp.abs(sb - float(k)), 0.0)
                    contribz[row, pl.ds(k * DH, DH)] = svec * ind

        pltpu.sync_copy(erows, eout.at[pl.ds(base, EB)])
        pltpu.sync_copy(contrib, acc.at[idx_sd.at[1]], add=True)
        pltpu.sync_copy(contribz, zacc.at[idx_z], add=True)
        return carry

    def block_body(b, carry):
        return do_block(wid * NBPW + b, carry)

    lax.fori_loop(0, NBPW, block_body, 0)

    @pl.when(cid == 0)
    def _():
        do_block(NW * NBPW + sid, 0)

    plsc.subcore_barrier()

    pltpu.sync_copy(acc.at[pl.ds(sid * RPT, RPT)],
                    accout.at[cid, pl.ds(sid * RPT, RPT)])
    pltpu.sync_copy(zacc.at[pl.ds(sid * ZPT, ZPT)],
                    zaccout.at[cid, pl.ds(sid * ZPT, ZPT)])


@functools.lru_cache(maxsize=1)
def _sc_edge_kernel():
    mesh = plsc.VectorSubcoreMesh(core_axis_name="c", subcore_axis_name="s",
                                  num_cores=NC, num_subcores=NS)
    return pl.kernel(
        _sc_edge_body,
        out_type=(jax.ShapeDtypeStruct((E, D), jnp.float32),
                  jax.ShapeDtypeStruct((NC, NP, D), jnp.float32),
                  jax.ShapeDtypeStruct((NC, NZ, D), jnp.float32)),
        mesh=mesh,
        scratch_types=[
            pltpu.VMEM((2, EB), jnp.int32),
            pltpu.VMEM((EB,), jnp.int32),
            pltpu.VMEM((16, D), jnp.float32),
            pltpu.VMEM((16, D), jnp.float32),
            pltpu.VMEM((16, D), jnp.float32),
            pltpu.VMEM((EB, D), jnp.float32),
            pltpu.VMEM((EB, D), jnp.float32),
            pltpu.VMEM((EB, D), jnp.float32),
            pltpu.VMEM((EB, D), jnp.float32),
            pltpu.VMEM((EB, D), jnp.float32),
            pltpu.VMEM_SHARED((NP, D), jnp.float32),
            pltpu.VMEM_SHARED((NZ, D), jnp.float32),
            pltpu.SemaphoreType.DMA,
        ],
    )


def _epost_body(eo_ref, e_ref, ow_ref, ob_ref, g1_ref, b1_ref,
                f1w_ref, f1b_ref, f2w_ref, f2b_ref, g2_ref, b2_ref, out_ref):
    x = _dot(eo_ref[...], ow_ref[...]) + ob_ref[...] + e_ref[...]
    x = _ln_rows(x, g1_ref[...], b1_ref[...])
    y = _dot(jnp.maximum(_dot(x, f1w_ref[...]) + f1b_ref[...], 0.0),
             f2w_ref[...]) + f2b_ref[...]
    out_ref[...] = _ln_rows(x + y, g2_ref[...], b2_ref[...])


def _epost(eo, e, ow, ob, g1, b1, f1w, f1b, f2w, f2b, g2, b2):
    blk = 2560
    grid = E // blk
    rspec = pl.BlockSpec((blk, D), lambda i: (i, 0))

    def fixed(shape):
        return pl.BlockSpec(shape, lambda i: tuple(0 for _ in shape))

    return pl.pallas_call(
        _epost_body,
        grid=(grid,),
        in_specs=[rspec, rspec,
                  fixed((D, D)), fixed((1, D)), fixed((1, D)), fixed((1, D)),
                  fixed((D, 2 * D)), fixed((1, 2 * D)),
                  fixed((2 * D, D)), fixed((1, D)),
                  fixed((1, D)), fixed((1, D))],
        out_specs=rspec,
        out_shape=jax.ShapeDtypeStruct((E, D), jnp.float32),
    )(eo, e, ow, ob, g1, b1, f1w, f1b, f2w, f2b, g2, b2)


def _hpost_body(acc_ref, z_ref, h_ref, b8_ref, ow_ref, ob_ref, g1_ref, b1_ref,
                f1w_ref, f1b_ref, f2w_ref, f2b_ref, g2_ref, b2_ref, out_ref):
    wv = acc_ref[0] + acc_ref[1]
    z16 = z_ref[0] + z_ref[1]
    zex = _dot(z16, b8_ref[...])
    hat = wv / (zex + 1e-6)
    x = _dot(hat, ow_ref[...]) + ob_ref[...] + h_ref[...]
    x = _ln_rows(x, g1_ref[...], b1_ref[...])
    y = _dot(jnp.maximum(_dot(x, f1w_ref[...]) + f1b_ref[...], 0.0),
             f2w_ref[...]) + f2b_ref[...]
    out_ref[...] = _ln_rows(x + y, g2_ref[...], b2_ref[...])


def _hpost(acc, zp, h, b8, ow, ob, g1, b1, f1w, f1b, f2w, f2b, g2, b2):
    blk = 2000
    grid = N // blk
    rspec = pl.BlockSpec((blk, D), lambda i: (i, 0))

    def fixed(shape):
        return pl.BlockSpec(shape, lambda i: tuple(0 for _ in shape))

    return pl.pallas_call(
        _hpost_body,
        grid=(grid,),
        in_specs=[pl.BlockSpec((NC, blk, D), lambda i: (0, i, 0)),
                  pl.BlockSpec((NC, blk, DH), lambda i: (0, i, 0)), rspec,
                  fixed((DH, D)),
                  fixed((D, D)), fixed((1, D)), fixed((1, D)), fixed((1, D)),
                  fixed((D, 2 * D)), fixed((1, 2 * D)),
                  fixed((2 * D, D)), fixed((1, D)),
                  fixed((1, D)), fixed((1, D))],
        out_specs=rspec,
        out_shape=jax.ShapeDtypeStruct((N, D), jnp.float32),
    )(acc, zp, h, b8, ow, ob, g1, b1, f1w, f1b, f2w, f2b, g2, b2)


def kernel(h, e, edge_index, Wq, Wk, Wv, We, Oh_W, Oh_b, Oe_W, Oe_b,
           ln1h_g, ln1h_b, ln1e_g, ln1e_b, F1h_W, F1h_b, F2h_W, F2h_b,
           F1e_W, F1e_b, F2e_W, F2e_b, ln2h_g, ln2h_b, ln2e_g, ln2e_b):
    src = edge_index[0]
    dst = edge_index[1]

    qh, kh, vh = _proj(h, Wq, Wk, Wv)
    pe = _pe(e, We)
    sdfr = jnp.stack([src.reshape(-1, EB), dst.reshape(-1, EB)], axis=1)
    eout, acc, zaccp = _sc_edge_kernel()(kh, qh, vh, pe, sdfr)
    zp = zaccp.reshape(NC, NP, DH)

    r = lambda v: v.reshape(1, -1)
    b8 = jnp.concatenate(
        [jnp.repeat(jnp.eye(H, dtype=jnp.float32), DH, axis=1),
         jnp.zeros((DH - H, D), jnp.float32)], axis=0)

    e3 = _epost(eout, e, Oe_W, r(Oe_b), r(ln1e_g), r(ln1e_b),
                F1e_W, r(F1e_b), F2e_W, r(F2e_b), r(ln2e_g), r(ln2e_b))
    h3 = _hpost(acc, zp, h, b8, Oh_W, r(Oh_b), r(ln1h_g), r(ln1h_b),
                F1h_W, r(F1h_b), F2h_W, r(F2h_b), r(ln2h_g), r(ln2h_b))
    return (h3, e3)


# speedup vs baseline: 1.1178x; 1.1178x over previous
import functools

import jax
import jax.numpy as jnp
import numpy as np
from jax import lax
from jax.experimental import pallas as pl
from jax.experimental.pallas import tpu as pltpu
from jax.experimental.pallas import tpu_sc as plsc

N = 10000
E = 320000
D = 128
H = 8
DH = D // H

NC = 2
NS = 16
NW = NC * NS
EPW = E // NW
EB = 16
NBLK = EPW // EB
NP = 10240
NZ = NP // 8
RPT = NP // NS
ZPT = NZ // NS

_PREC = lax.Precision.HIGHEST


def _dot(a, b):
    return jnp.dot(a, b, preferred_element_type=jnp.float32, precision=_PREC)


def _ln_rows(x, g, b):
    mu = jnp.mean(x, axis=-1, keepdims=True)
    xc = x - mu
    var = jnp.mean(xc * xc, axis=-1, keepdims=True)
    return xc * lax.rsqrt(var + 1e-5) * g + b


def _proj_body(h_ref, wq_ref, wk_ref, wv_ref, q_ref, k_ref, v_ref):
    hb = h_ref[...]
    q_ref[...] = _dot(hb, wq_ref[...])
    k_ref[...] = _dot(hb, wk_ref[...]) * (1.0 / np.sqrt(DH))
    v_ref[...] = _dot(hb, wv_ref[...])


def _proj(h, wq, wk, wv):
    blk = 2000
    grid = N // blk
    wspec = pl.BlockSpec((D, D), lambda i: (0, 0))
    rspec = pl.BlockSpec((blk, D), lambda i: (i, 0))
    return pl.pallas_call(
        _proj_body,
        grid=(grid,),
        in_specs=[rspec, wspec, wspec, wspec],
        out_specs=[rspec, rspec, rspec],
        out_shape=[jax.ShapeDtypeStruct((N, D), jnp.float32)] * 3,
    )(h, wq, wk, wv)


def _pe_body(e_ref, we_ref, pe_ref):
    pe_ref[...] = _dot(e_ref[...], we_ref[...])


def _pe(e, we):
    blk = 2560
    grid = E // blk
    return pl.pallas_call(
        _pe_body,
        grid=(grid,),
        in_specs=[pl.BlockSpec((blk, D), lambda i: (i, 0)),
                  pl.BlockSpec((D, D), lambda i: (0, 0))],
        out_specs=pl.BlockSpec((blk, D), lambda i: (i, 0)),
        out_shape=jax.ShapeDtypeStruct((E, D), jnp.float32),
    )(e, we)


def _sc_edge_body(kh, qh, vh, pe, srci, dsti, eout, accout, zaccout,
                  idx_s, idx_d, idx_z, krows, qrows, vrows, perows, erows,
                  contrib, contribz, zbuf, acc, zacc, sem):
    cid = lax.axis_index("c")
    sid = lax.axis_index("s")
    wid = cid * NS + sid

    zv = jnp.zeros((16,), jnp.float32)
    lanes = lax.iota(jnp.int32, 16)

    def zbrow(i, carry):
        for j in range(D // 16):
            zbuf[i, pl.ds(j * 16, 16)] = zv
        return carry

    lax.fori_loop(0, EB, zbrow, 0)

    def zaccrow(b, carry):
        pltpu.sync_copy(zbuf, acc.at[pl.ds(sid * RPT + b * EB, EB)])
        return carry

    lax.fori_loop(0, RPT // EB, zaccrow, 0)

    def zzrow(b, carry):
        pltpu.sync_copy(zbuf, zacc.at[pl.ds(sid * ZPT + b * EB, EB)])
        return carry

    lax.fori_loop(0, ZPT // EB, zzrow, 0)
    plsc.subcore_barrier()

    gdn = lax.GatherDimensionNumbers(offset_dims=(), collapsed_slice_dims=(0,),
                                     start_index_map=(0,))
    bfly_idx = [(lanes ^ k)[:, None] for k in (8, 4, 2, 1)]

    def _lane_allsum(v):
        for idx in bfly_idx:
            v = v + lax.gather(v, idx, gdn, slice_sizes=(1,),
                               mode=lax.GatherScatterMode.PROMISE_IN_BOUNDS)
        return v

    ebase = wid * EPW

    def block_body(b, carry):
        base = ebase + b * EB
        pltpu.sync_copy(srci.at[pl.ds(base, EB)], idx_s)
        pltpu.sync_copy(dsti.at[pl.ds(base, EB)], idx_d)
        c1 = pltpu.async_copy(kh.at[idx_s], krows, sem)
        c2 = pltpu.async_copy(qh.at[idx_d], qrows, sem)
        c3 = pltpu.async_copy(vh.at[idx_s], vrows, sem)
        pltpu.sync_copy(pe.at[pl.ds(base, EB)], perows)

        dv = idx_d[pl.ds(0, 16)]
        idx_z[pl.ds(0, 16)] = lax.shift_right_logical(dv, 3)
        c1.wait()
        c2.wait()
        c3.wait()

        slotf = lax.bitwise_and(dv, 7).astype(jnp.float32)
        for i in range(EB):
            svec = zv
            for hh in range(H):
                sl = pl.ds(hh * DH, DH)
                sc = krows[i, sl] * qrows[i, sl] * perows[i, sl]
                erows[i, sl] = sc
                tot = _lane_allsum(sc)
                sv = jnp.exp(jnp.minimum(jnp.maximum(tot, -5.0), 5.0))
                contrib[i, sl] = vrows[i, sl] * sv
                svec = jnp.where(lanes == hh, sv, svec)
            sb = _lane_allsum(jnp.where(lanes == i, slotf, 0.0))
            for k in range(8):
                ind = jnp.maximum(1.0 - jnp.abs(sb - float(k)), 0.0)
                contribz[i, pl.ds(k * DH, DH)] = svec * ind

        pltpu.sync_copy(erows, eout.at[pl.ds(base, EB)])
        pltpu.sync_copy(contrib, acc.at[idx_d], add=True)
        pltpu.sync_copy(contribz, zacc.at[idx_z], add=True)
        return carry

    lax.fori_loop(0, NBLK, block_body, 0)

    plsc.subcore_barrier()

    pltpu.sync_copy(acc.at[pl.ds(sid * RPT, RPT)],
                    accout.at[cid, pl.ds(sid * RPT, RPT)])
    pltpu.sync_copy(zacc.at[pl.ds(sid * ZPT, ZPT)],
                    zaccout.at[cid, pl.ds(sid * ZPT, ZPT)])


@functools.lru_cache(maxsize=1)
def _sc_edge_kernel():
    mesh = plsc.VectorSubcoreMesh(core_axis_name="c", subcore_axis_name="s",
                                  num_cores=NC, num_subcores=NS)
    return pl.kernel(
        _sc_edge_body,
        out_type=(jax.ShapeDtypeStruct((E, D), jnp.float32),
                  jax.ShapeDtypeStruct((NC, NP, D), jnp.float32),
                  jax.ShapeDtypeStruct((NC, NZ, D), jnp.float32)),
        mesh=mesh,
        scratch_types=[
            pltpu.VMEM((EB,), jnp.int32),
            pltpu.VMEM((EB,), jnp.int32),
            pltpu.VMEM((EB,), jnp.int32),
            pltpu.VMEM((EB, D), jnp.float32),
            pltpu.VMEM((EB, D), jnp.float32),
            pltpu.VMEM((EB, D), jnp.float32),
            pltpu.VMEM((EB, D), jnp.float32),
            pltpu.VMEM((EB, D), jnp.float32),
            pltpu.VMEM((EB, D), jnp.float32),
            pltpu.VMEM((EB, D), jnp.float32),
            pltpu.VMEM((EB, D), jnp.float32),
            pltpu.VMEM_SHARED((NP, D), jnp.float32),
            pltpu.VMEM_SHARED((NZ, D), jnp.float32),
            pltpu.SemaphoreType.DMA,
        ],
    )


def _epost_body(eo_ref, e_ref, ow_ref, ob_ref, g1_ref, b1_ref,
                f1w_ref, f1b_ref, f2w_ref, f2b_ref, g2_ref, b2_ref, out_ref):
    x = _dot(eo_ref[...], ow_ref[...]) + ob_ref[...] + e_ref[...]
    x = _ln_rows(x, g1_ref[...], b1_ref[...])
    y = _dot(jnp.maximum(_dot(x, f1w_ref[...]) + f1b_ref[...], 0.0),
             f2w_ref[...]) + f2b_ref[...]
    out_ref[...] = _ln_rows(x + y, g2_ref[...], b2_ref[...])


def _epost(eo, e, ow, ob, g1, b1, f1w, f1b, f2w, f2b, g2, b2):
    blk = 2560
    grid = E // blk
    rspec = pl.BlockSpec((blk, D), lambda i: (i, 0))

    def fixed(shape):
        return pl.BlockSpec(shape, lambda i: tuple(0 for _ in shape))

    return pl.pallas_call(
        _epost_body,
        grid=(grid,),
        in_specs=[rspec, rspec,
                  fixed((D, D)), fixed((1, D)), fixed((1, D)), fixed((1, D)),
                  fixed((D, 2 * D)), fixed((1, 2 * D)),
                  fixed((2 * D, D)), fixed((1, D)),
                  fixed((1, D)), fixed((1, D))],
        out_specs=rspec,
        out_shape=jax.ShapeDtypeStruct((E, D), jnp.float32),
    )(eo, e, ow, ob, g1, b1, f1w, f1b, f2w, f2b, g2, b2)


def _hpost_body(acc_ref, z_ref, h_ref, b8_ref, ow_ref, ob_ref, g1_ref, b1_ref,
                f1w_ref, f1b_ref, f2w_ref, f2b_ref, g2_ref, b2_ref, out_ref):
    wv = acc_ref[0] + acc_ref[1]
    z16 = z_ref[0] + z_ref[1]
    zex = _dot(z16, b8_ref[...])
    hat = wv / (zex + 1e-6)
    x = _dot(hat, ow_ref[...]) + ob_ref[...] + h_ref[...]
    x = _ln_rows(x, g1_ref[...], b1_ref[...])
    y = _dot(jnp.maximum(_dot(x, f1w_ref[...]) + f1b_ref[...], 0.0),
             f2w_ref[...]) + f2b_ref[...]
    out_ref[...] = _ln_rows(x + y, g2_ref[...], b2_ref[...])


def _hpost(acc, zp, h, b8, ow, ob, g1, b1, f1w, f1b, f2w, f2b, g2, b2):
    blk = 2000
    grid = N // blk
    rspec = pl.BlockSpec((blk, D), lambda i: (i, 0))

    def fixed(shape):
        return pl.BlockSpec(shape, lambda i: tuple(0 for _ in shape))

    return pl.pallas_call(
        _hpost_body,
        grid=(grid,),
        in_specs=[pl.BlockSpec((NC, blk, D), lambda i: (0, i, 0)),
                  pl.BlockSpec((NC, blk, DH), lambda i: (0, i, 0)), rspec,
                  fixed((DH, D)),
                  fixed((D, D)), fixed((1, D)), fixed((1, D)), fixed((1, D)),
                  fixed((D, 2 * D)), fixed((1, 2 * D)),
                  fixed((2 * D, D)), fixed((1, D)),
                  fixed((1, D)), fixed((1, D))],
        out_specs=rspec,
        out_shape=jax.ShapeDtypeStruct((N, D), jnp.float32),
    )(acc, zp, h, b8, ow, ob, g1, b1, f1w, f1b, f2w, f2b, g2, b2)


def kernel(h, e, edge_index, Wq, Wk, Wv, We, Oh_W, Oh_b, Oe_W, Oe_b,
           ln1h_g, ln1h_b, ln1e_g, ln1e_b, F1h_W, F1h_b, F2h_W, F2h_b,
           F1e_W, F1e_b, F2e_W, F2e_b, ln2h_g, ln2h_b, ln2e_g, ln2e_b):
    src = edge_index[0]
    dst = edge_index[1]

    qh, kh, vh = _proj(h, Wq, Wk, Wv)
    pe = _pe(e, We)
    eout, acc, zaccp = _sc_edge_kernel()(kh, qh, vh, pe, src, dst)
    zp = zaccp.reshape(NC, NP, DH)

    r = lambda v: v.reshape(1, -1)
    b8 = jnp.concatenate(
        [jnp.repeat(jnp.eye(H, dtype=jnp.float32), DH, axis=1),
         jnp.zeros((DH - H, D), jnp.float32)], axis=0)

    e3 = _epost(eout, e, Oe_W, r(Oe_b), r(ln1e_g), r(ln1e_b),
                F1e_W, r(F1e_b), F2e_W, r(F2e_b), r(ln2e_g), r(ln2e_b))
    h3 = _hpost(acc, zp, h, b8, Oh_W, r(Oh_b), r(ln1h_g), r(ln1h_b),
                F1h_W, r(F1h_b), F2h_W, r(F2h_b), r(ln2h_g), r(ln2h_b))
    return (h3, e3)
